# baseline (device time: 56800 ns/iter reference)
import jax
import jax.numpy as jnp
from jax import lax
from jax.experimental import pallas as pl
from jax.experimental.pallas import tpu as pltpu

N_DEV = 4
NSUB = 3


def kernel(A, B):
    m, k = A.shape
    k2, n = B.shape
    C = m // N_DEV
    H = n // 2
    R = C // NSUB

    def body(a_ref, b_ref, out_ref, a_v, b_v,
             cw_send, cw_recv, ccw_send, ccw_recv, ag_cw, ag_ccw,
             cw_ssem, cw_rsem, ccw_ssem, ccw_rsem,
             agcw_ssem, agcw_rsem, agccw_ssem, agccw_rsem,
             in_sems):
        d = lax.axis_index("i")
        left = (d + N_DEV - 1) % N_DEV
        right = (d + 1) % N_DEV

        ca = [(d + N_DEV - 1) % N_DEV, (d + 1) % N_DEV,
              (d + 2) % N_DEV, d]
        in_cps = []
        for i, c in enumerate(ca):
            rows = pl.ds(c * C, C)
            cp = pltpu.make_async_copy(a_ref.at[rows, :], a_v.at[rows, :],
                                       in_sems.at[i])
            cp.start()
            in_cps.append(cp)
            if i < 2:
                cols = pl.ds(i * H, H)
                cpb = pltpu.make_async_copy(b_ref.at[:, cols],
                                            b_v.at[:, cols], in_sems.at[4 + i])
                cpb.start()
                in_cps.append(cpb)

        barrier_sem = pltpu.get_barrier_semaphore()
        for nbr in (left, right):
            pl.semaphore_signal(
                barrier_sem, inc=1,
                device_id=(nbr,), device_id_type=pl.DeviceIdType.MESH,
            )
        pl.semaphore_wait(barrier_sem, 2)

        f32 = jnp.float32
        bf16 = jnp.bfloat16

        def p_left(c, j):
            return jnp.dot(a_v[pl.ds(c * C + j * R, R), :], b_v[:, :H],
                           preferred_element_type=f32)

        def p_right(c, j):
            return jnp.dot(a_v[pl.ds(c * C + j * R, R), :], b_v[:, H:],
                           preferred_element_type=f32)

        def rdma(buf_s, buf_r, slot_s, slot_r, j, ssem, rsem, dev):
            return pltpu.make_async_remote_copy(
                src_ref=buf_s.at[slot_s, pl.ds(j * R, R), :],
                dst_ref=buf_r.at[slot_r, pl.ds(j * R, R), :],
                send_sem=ssem.at[slot_s, j], recv_sem=rsem.at[slot_s, j],
                device_id=(dev,), device_id_type=pl.DeviceIdType.MESH,
            )

        rs_cw = [[rdma(cw_send, cw_recv, s, s, j, cw_ssem, cw_rsem, right)
                  for j in range(NSUB)] for s in range(N_DEV - 1)]
        rs_ccw = [[rdma(ccw_send, ccw_recv, s, s, j, ccw_ssem, ccw_rsem, left)
                   for j in range(NSUB)] for s in range(N_DEV - 1)]
        ag_cws = [[rdma(ag_cw, ag_cw, h, h + 1, j, agcw_ssem, agcw_rsem,
                        right) for j in range(NSUB)] for h in range(N_DEV - 1)]
        ag_ccws = [[rdma(ag_ccw, ag_ccw, h, h + 1, j, agccw_ssem, agccw_rsem,
                         left) for j in range(NSUB)] for h in range(N_DEV - 1)]

        in_cps[0].wait()
        in_cps[1].wait()
        for j in range(NSUB):
            cw_send[0, pl.ds(j * R, R), :] = p_left(ca[0], j).astype(bf16)
            rs_cw[0][j].start()
        in_cps[2].wait()
        in_cps[3].wait()
        for j in range(NSUB):
            ccw_send[0, pl.ds(j * R, R), :] = p_right(ca[1], j).astype(bf16)
            rs_ccw[0][j].start()

        for s in range(N_DEV - 2):
            c_cw = (d + 2 * N_DEV - 2 - s) % N_DEV
            c_ccw = (d + 2 + s) % N_DEV
            in_cps[4 + s].wait()
            pcw = [p_left(c_cw, j) for j in range(NSUB)]
            pccw = [p_right(c_ccw, j) for j in range(NSUB)]
            for j in range(NSUB):
                sub = pl.ds(j * R, R)
                rs_cw[s][j].wait_recv()
                cw_send[s + 1, sub, :] = (
                    pcw[j] + cw_recv[s, sub, :].astype(f32)).astype(bf16)
                rs_cw[s + 1][j].start()
                rs_ccw[s][j].wait_recv()
                ccw_send[s + 1, sub, :] = (
                    pccw[j] + ccw_recv[s, sub, :].astype(f32)).astype(bf16)
                rs_ccw[s + 1][j].start()

        last = N_DEV - 2
        pf = [jnp.dot(a_v[pl.ds(d * C + j * R, R), :], b_v[...],
                      preferred_element_type=f32) for j in range(NSUB)]
        for j in range(NSUB):
            sub = pl.ds(j * R, R)
            rs_cw[last][j].wait_recv()
            lh = jnp.maximum(pf[j][:, :H] + cw_recv[last, sub, :].astype(f32),
                             0.0)
            ag_cw[0, sub, :] = lh.astype(bf16)
            ag_cws[0][j].start()
            rs_ccw[last][j].wait_recv()
            rh = jnp.maximum(pf[j][:, H:] + ccw_recv[last, sub, :].astype(f32),
                             0.0)
            ag_ccw[0, sub, :] = rh.astype(bf16)
            ag_ccws[0][j].start()
            out_ref[pl.ds(d * C + j * R, R), :H] = lh
            out_ref[pl.ds(d * C + j * R, R), H:] = rh

        for h in range(N_DEV - 1):
            o_cw = (d + N_DEV - 1 - h) % N_DEV
            o_ccw = (d + 1 + h) % N_DEV
            for j in range(NSUB):
                sub = pl.ds(j * R, R)
                ag_cws[h][j].wait_recv()
                if h < N_DEV - 2:
                    ag_cws[h + 1][j].start()
                ag_ccws[h][j].wait_recv()
                if h < N_DEV - 2:
                    ag_ccws[h + 1][j].start()
                out_ref[pl.ds(o_cw * C + j * R, R), :H] = (
                    ag_cw[h + 1, sub, :].astype(f32))
                out_ref[pl.ds(o_ccw * C + j * R, R), H:] = (
                    ag_ccw[h + 1, sub, :].astype(f32))

        for group in (rs_cw, rs_ccw, ag_cws, ag_ccws):
            for pair in group:
                for r in pair:
                    r.wait_send()

    sem2 = pltpu.SemaphoreType.DMA((N_DEV - 1, NSUB))
    hbm = pl.BlockSpec(memory_space=pltpu.MemorySpace.HBM)
    return pl.pallas_call(
        body,
        out_shape=jax.ShapeDtypeStruct((m, n), jnp.float32),
        in_specs=[hbm, hbm],
        out_specs=pl.BlockSpec(memory_space=pltpu.VMEM),
        scratch_shapes=[
            pltpu.VMEM((m, k), jnp.float32),
            pltpu.VMEM((k, n), jnp.float32),
            pltpu.VMEM((N_DEV - 1, C, H), jnp.bfloat16),
            pltpu.VMEM((N_DEV - 1, C, H), jnp.bfloat16),
            pltpu.VMEM((N_DEV - 1, C, H), jnp.bfloat16),
            pltpu.VMEM((N_DEV - 1, C, H), jnp.bfloat16),
            pltpu.VMEM((N_DEV, C, H), jnp.bfloat16),
            pltpu.VMEM((N_DEV, C, H), jnp.bfloat16),
            sem2,
            sem2,
            sem2,
            sem2,
            sem2,
            sem2,
            sem2,
            sem2,
            pltpu.SemaphoreType.DMA((6,)),
        ],
        compiler_params=pltpu.CompilerParams(
            collective_id=0, vmem_limit_bytes=64 * 1024 * 1024),
    )(A, B)


# device time: 52706 ns/iter; 1.0777x vs baseline; 1.0777x over previous
import jax
import jax.numpy as jnp
from jax import lax
from jax.experimental import pallas as pl
from jax.experimental.pallas import tpu as pltpu

N_DEV = 4
NSUB = 3


def kernel(A, B):
    m, k = A.shape
    k2, n = B.shape
    C = m // N_DEV
    H = n // 2
    R = C // NSUB

    def body(a_ref, b_ref, out_ref,
             cw_send, cw_recv, ccw_send, ccw_recv, ag_cw, ag_ccw,
             cw_ssem, cw_rsem, ccw_ssem, ccw_rsem,
             agcw_ssem, agcw_rsem, agccw_ssem, agccw_rsem):
        d = lax.axis_index("i")
        left = (d + N_DEV - 1) % N_DEV
        right = (d + 1) % N_DEV

        barrier_sem = pltpu.get_barrier_semaphore()
        for nbr in (left, right):
            pl.semaphore_signal(
                barrier_sem, inc=1,
                device_id=(nbr,), device_id_type=pl.DeviceIdType.MESH,
            )
        pl.semaphore_wait(barrier_sem, 2)

        f32 = jnp.float32
        bf16 = jnp.bfloat16

        def p_left(c, j):
            return jnp.dot(a_ref[pl.ds(c * C + j * R, R), :], b_ref[:, :H],
                           preferred_element_type=f32)

        def p_right(c, j):
            return jnp.dot(a_ref[pl.ds(c * C + j * R, R), :], b_ref[:, H:],
                           preferred_element_type=f32)

        def rdma(buf_s, buf_r, slot_s, slot_r, j, ssem, rsem, dev):
            return pltpu.make_async_remote_copy(
                src_ref=buf_s.at[slot_s, pl.ds(j * R, R), :],
                dst_ref=buf_r.at[slot_r, pl.ds(j * R, R), :],
                send_sem=ssem.at[slot_s, j], recv_sem=rsem.at[slot_s, j],
                device_id=(dev,), device_id_type=pl.DeviceIdType.MESH,
            )

        rs_cw = [[rdma(cw_send, cw_recv, s, s, j, cw_ssem, cw_rsem, right)
                  for j in range(NSUB)] for s in range(N_DEV - 1)]
        rs_ccw = [[rdma(ccw_send, ccw_recv, s, s, j, ccw_ssem, ccw_rsem, left)
                   for j in range(NSUB)] for s in range(N_DEV - 1)]
        ag_cws = [[rdma(ag_cw, ag_cw, h, h + 1, j, agcw_ssem, agcw_rsem,
                        right) for j in range(NSUB)] for h in range(N_DEV - 1)]
        ag_ccws = [[rdma(ag_ccw, ag_ccw, h, h + 1, j, agccw_ssem, agccw_rsem,
                         left) for j in range(NSUB)] for h in range(N_DEV - 1)]

        c_cw0 = (d + N_DEV - 1) % N_DEV
        c_ccw0 = (d + 1) % N_DEV
        for j in range(NSUB):
            cw_send[0, pl.ds(j * R, R), :] = p_left(c_cw0, j).astype(bf16)
            rs_cw[0][j].start()
            ccw_send[0, pl.ds(j * R, R), :] = p_right(c_ccw0, j).astype(bf16)
            rs_ccw[0][j].start()

        for s in range(N_DEV - 2):
            c_cw = (d + 2 * N_DEV - 2 - s) % N_DEV
            c_ccw = (d + 2 + s) % N_DEV
            pcw = [p_left(c_cw, j) for j in range(NSUB)]
            pccw = [p_right(c_ccw, j) for j in range(NSUB)]
            for j in range(NSUB):
                sub = pl.ds(j * R, R)
                rs_cw[s][j].wait_recv()
                cw_send[s + 1, sub, :] = (
                    pcw[j] + cw_recv[s, sub, :].astype(f32)).astype(bf16)
                rs_cw[s + 1][j].start()
                rs_ccw[s][j].wait_recv()
                ccw_send[s + 1, sub, :] = (
                    pccw[j] + ccw_recv[s, sub, :].astype(f32)).astype(bf16)
                rs_ccw[s + 1][j].start()

        last = N_DEV - 2
        pf = [jnp.dot(a_ref[pl.ds(d * C + j * R, R), :], b_ref[...],
                      preferred_element_type=f32) for j in range(NSUB)]
        for j in range(NSUB):
            sub = pl.ds(j * R, R)
            rs_cw[last][j].wait_recv()
            lh = jnp.maximum(pf[j][:, :H] + cw_recv[last, sub, :].astype(f32),
                             0.0)
            ag_cw[0, sub, :] = lh.astype(bf16)
            ag_cws[0][j].start()
            rs_ccw[last][j].wait_recv()
            rh = jnp.maximum(pf[j][:, H:] + ccw_recv[last, sub, :].astype(f32),
                             0.0)
            ag_ccw[0, sub, :] = rh.astype(bf16)
            ag_ccws[0][j].start()
            out_ref[pl.ds(d * C + j * R, R), :H] = lh
            out_ref[pl.ds(d * C + j * R, R), H:] = rh

        for h in range(N_DEV - 1):
            o_cw = (d + N_DEV - 1 - h) % N_DEV
            o_ccw = (d + 1 + h) % N_DEV
            for j in range(NSUB):
                sub = pl.ds(j * R, R)
                ag_cws[h][j].wait_recv()
                if h < N_DEV - 2:
                    ag_cws[h + 1][j].start()
                ag_ccws[h][j].wait_recv()
                if h < N_DEV - 2:
                    ag_ccws[h + 1][j].start()
                out_ref[pl.ds(o_cw * C + j * R, R), :H] = (
                    ag_cw[h + 1, sub, :].astype(f32))
                out_ref[pl.ds(o_ccw * C + j * R, R), H:] = (
                    ag_ccw[h + 1, sub, :].astype(f32))

        for group in (rs_cw, rs_ccw, ag_cws, ag_ccws):
            for pair in group:
                for r in pair:
                    r.wait_send()

    sem2 = pltpu.SemaphoreType.DMA((N_DEV - 1, NSUB))
    return pl.pallas_call(
        body,
        out_shape=jax.ShapeDtypeStruct((m, n), jnp.float32),
        in_specs=[
            pl.BlockSpec(memory_space=pltpu.VMEM),
            pl.BlockSpec(memory_space=pltpu.VMEM),
        ],
        out_specs=pl.BlockSpec(memory_space=pltpu.VMEM),
        scratch_shapes=[
            pltpu.VMEM((N_DEV - 1, C, H), jnp.bfloat16),
            pltpu.VMEM((N_DEV - 1, C, H), jnp.bfloat16),
            pltpu.VMEM((N_DEV - 1, C, H), jnp.bfloat16),
            pltpu.VMEM((N_DEV - 1, C, H), jnp.bfloat16),
            pltpu.VMEM((N_DEV, C, H), jnp.bfloat16),
            pltpu.VMEM((N_DEV, C, H), jnp.bfloat16),
            sem2,
            sem2,
            sem2,
            sem2,
            sem2,
            sem2,
            sem2,
            sem2,
        ],
        compiler_params=pltpu.CompilerParams(collective_id=0),
    )(A, B)
